# Initial kernel scaffold; baseline (speedup 1.0000x reference)
import jax
import jax.numpy as jnp
from jax.experimental import pallas as pl


def kernel(x, E1, E2, E3, W1, b1, W2, b2):
    raise NotImplementedError("write your pallas kernel here")


# speedup vs baseline: 14.4825x; 14.4825x over previous
import functools

import jax
import jax.numpy as jnp
from jax import lax
from jax.experimental import pallas as pl
from jax.experimental.pallas import tpu as pltpu
from jax.experimental.pallas import tpu_sc as plsc

BATCH = 4096
SEQ = 200
EMBED = 64
FEAT = 3 * EMBED
HIDDEN = 256
NUM_CLASSES = 10
OUT_PAD = 128

_NC = 2
_NS = 16
_NW = _NC * _NS
_RPW = BATCH // _NW

_CHUNKS = ((0, 128), (128, 72))


def _pool_body(x_hbm, e1_hbm, e2_hbm, e3_hbm, out_hbm, xv, b1v, b2v, b3v,
               outv, sem):
    wid = lax.axis_index("s") * _NC + lax.axis_index("c")
    base = wid * _RPW
    pltpu.sync_copy(x_hbm.at[pl.ds(base, _RPW)], xv)

    tables = (e1_hbm, e2_hbm, e3_hbm)
    bufs = (b1v, b2v, b3v)
    inv_seq = 1.0 / SEQ

    def row_step(r, carry):
        copies = []
        for t in range(3):
            for (off, cnt) in _CHUNKS:
                copies.append(pltpu.async_copy(
                    tables[t].at[xv.at[r, pl.ds(off, cnt)]],
                    bufs[t].at[pl.ds(off, cnt)],
                    sem))
        for c in copies:
            c.wait()

        def acc_step(s, accs):
            new = []
            for t in range(3):
                for j in range(4):
                    new.append(accs[4 * t + j] + bufs[t][s, pl.ds(16 * j, 16)])
            return tuple(new)

        zero = jnp.zeros((16,), jnp.float32)
        accs = lax.fori_loop(0, SEQ, acc_step, (zero,) * 12)
        for t in range(3):
            for j in range(4):
                outv[r, pl.ds(64 * t + 16 * j, 16)] = accs[4 * t + j] * inv_seq
        return carry

    lax.fori_loop(0, _RPW, row_step, 0)
    pltpu.sync_copy(outv, out_hbm.at[pl.ds(base, _RPW)])


@jax.jit
def _pool(x, E1, E2, E3):
    mesh = plsc.VectorSubcoreMesh(core_axis_name="c", subcore_axis_name="s")
    return pl.kernel(
        _pool_body,
        out_type=jax.ShapeDtypeStruct((BATCH, FEAT), jnp.float32),
        mesh=mesh,
        scratch_types=[
            pltpu.VMEM((_RPW, SEQ), jnp.int32),
            pltpu.VMEM((SEQ, EMBED), jnp.float32),
            pltpu.VMEM((SEQ, EMBED), jnp.float32),
            pltpu.VMEM((SEQ, EMBED), jnp.float32),
            pltpu.VMEM((_RPW, FEAT), jnp.float32),
            pltpu.SemaphoreType.DMA,
        ],
        compiler_params=pltpu.CompilerParams(use_tc_tiling_on_sc=False),
    )(x, E1, E2, E3)


def _mlp_body(p_ref, w1_ref, b1_ref, w2_ref, b2_ref, o_ref):
    h = jnp.dot(p_ref[...], w1_ref[...], preferred_element_type=jnp.float32)
    h = jnp.maximum(h + b1_ref[...], 0.0)
    o_ref[...] = (
        jnp.dot(h, w2_ref[...], preferred_element_type=jnp.float32)
        + b2_ref[...])


@jax.jit
def _mlp(pooled, W1, b1, W2p, b2p):
    blk = 256
    grid = BATCH // blk
    return pl.pallas_call(
        _mlp_body,
        grid=(grid,),
        in_specs=[
            pl.BlockSpec((blk, FEAT), lambda i: (i, 0)),
            pl.BlockSpec((FEAT, HIDDEN), lambda i: (0, 0)),
            pl.BlockSpec((1, HIDDEN), lambda i: (0, 0)),
            pl.BlockSpec((HIDDEN, OUT_PAD), lambda i: (0, 0)),
            pl.BlockSpec((1, OUT_PAD), lambda i: (0, 0)),
        ],
        out_specs=pl.BlockSpec((blk, OUT_PAD), lambda i: (i, 0)),
        out_shape=jax.ShapeDtypeStruct((BATCH, OUT_PAD), jnp.float32),
    )(pooled, W1, b1, W2p, b2p)


def kernel(x, E1, E2, E3, W1, b1, W2, b2):
    pooled = _pool(x.astype(jnp.int32), E1, E2, E3)
    W2p = jnp.pad(W2, ((0, 0), (0, OUT_PAD - NUM_CLASSES)))
    b2p = jnp.pad(b2, (0, OUT_PAD - NUM_CLASSES)).reshape(1, OUT_PAD)
    out = _mlp(pooled, W1, b1.reshape(1, HIDDEN), W2p, b2p)
    return out[:, :NUM_CLASSES]


# speedup vs baseline: 19.4431x; 1.3425x over previous
import functools

import jax
import jax.numpy as jnp
from jax import lax
from jax.experimental import pallas as pl
from jax.experimental.pallas import tpu as pltpu
from jax.experimental.pallas import tpu_sc as plsc

BATCH = 4096
SEQ = 200
EMBED = 64
FEAT = 3 * EMBED
HIDDEN = 256
NUM_CLASSES = 10
OUT_PAD = 128

_NC = 2
_NS = 16
_NW = _NC * _NS
_RPW = BATCH // _NW

_CHUNKS = ((0, 128), (128, 72))


def _pool_body(x_hbm, e1_hbm, e2_hbm, e3_hbm, out_hbm, xv,
               b1v0, b2v0, b3v0, b1v1, b2v1, b3v1, outv, sem0, sem1):
    wid = lax.axis_index("s") * _NC + lax.axis_index("c")
    base = wid * _RPW
    pltpu.sync_copy(x_hbm.at[pl.ds(base, _RPW)], xv)

    tables = (e1_hbm, e2_hbm, e3_hbm)
    bufs = ((b1v0, b2v0, b3v0), (b1v1, b2v1, b3v1))
    sems = (sem0, sem1)
    inv_seq = 1.0 / SEQ

    def copies(r, slot):
        return [
            pltpu.make_async_copy(
                tables[t].at[xv.at[r, pl.ds(off, cnt)]],
                bufs[slot][t].at[pl.ds(off, cnt)],
                sems[slot])
            for t in range(3) for (off, cnt) in _CHUNKS
        ]

    def fire(r, slot):
        for c in copies(r, slot):
            c.start()

    def drain(r, slot):
        for c in copies(r, slot):
            c.wait()

    def accumulate(r, slot):
        bs = bufs[slot]

        def acc_step(i, accs):
            s = 2 * i
            new = list(accs)
            for u in range(2):
                for t in range(3):
                    for j in range(4):
                        new[4 * t + j] = (
                            new[4 * t + j] + bs[t][s + u, pl.ds(16 * j, 16)])
            return tuple(new)

        zero = jnp.zeros((16,), jnp.float32)
        accs = lax.fori_loop(0, SEQ // 2, acc_step, (zero,) * 12)
        for t in range(3):
            for j in range(4):
                outv[r, pl.ds(64 * t + 16 * j, 16)] = accs[4 * t + j] * inv_seq

    fire(0, 0)

    def pair_step(i, carry):
        r0 = 2 * i
        drain(r0, 0)
        fire(r0 + 1, 1)
        accumulate(r0, 0)
        drain(r0 + 1, 1)

        @pl.when(i < _RPW // 2 - 1)
        def _():
            fire(r0 + 2, 0)

        accumulate(r0 + 1, 1)
        return carry

    lax.fori_loop(0, _RPW // 2, pair_step, 0)
    pltpu.sync_copy(outv, out_hbm.at[pl.ds(base, _RPW)])


@jax.jit
def _pool(x, E1, E2, E3):
    mesh = plsc.VectorSubcoreMesh(core_axis_name="c", subcore_axis_name="s")
    return pl.kernel(
        _pool_body,
        out_type=jax.ShapeDtypeStruct((BATCH, FEAT), jnp.float32),
        mesh=mesh,
        scratch_types=[
            pltpu.VMEM((_RPW, SEQ), jnp.int32),
            pltpu.VMEM((SEQ, EMBED), jnp.float32),
            pltpu.VMEM((SEQ, EMBED), jnp.float32),
            pltpu.VMEM((SEQ, EMBED), jnp.float32),
            pltpu.VMEM((SEQ, EMBED), jnp.float32),
            pltpu.VMEM((SEQ, EMBED), jnp.float32),
            pltpu.VMEM((SEQ, EMBED), jnp.float32),
            pltpu.VMEM((_RPW, FEAT), jnp.float32),
            pltpu.SemaphoreType.DMA,
            pltpu.SemaphoreType.DMA,
        ],
        compiler_params=pltpu.CompilerParams(use_tc_tiling_on_sc=False),
    )(x, E1, E2, E3)


def _mlp_body(p_ref, w1_ref, b1_ref, w2_ref, b2_ref, o_ref):
    h = jnp.dot(p_ref[...], w1_ref[...], preferred_element_type=jnp.float32)
    h = jnp.maximum(h + b1_ref[...], 0.0)
    o_ref[...] = (
        jnp.dot(h, w2_ref[...], preferred_element_type=jnp.float32)
        + b2_ref[...])


@jax.jit
def _mlp(pooled, W1, b1, W2p, b2p):
    blk = 256
    grid = BATCH // blk
    return pl.pallas_call(
        _mlp_body,
        grid=(grid,),
        in_specs=[
            pl.BlockSpec((blk, FEAT), lambda i: (i, 0)),
            pl.BlockSpec((FEAT, HIDDEN), lambda i: (0, 0)),
            pl.BlockSpec((1, HIDDEN), lambda i: (0, 0)),
            pl.BlockSpec((HIDDEN, OUT_PAD), lambda i: (0, 0)),
            pl.BlockSpec((1, OUT_PAD), lambda i: (0, 0)),
        ],
        out_specs=pl.BlockSpec((blk, OUT_PAD), lambda i: (i, 0)),
        out_shape=jax.ShapeDtypeStruct((BATCH, OUT_PAD), jnp.float32),
    )(pooled, W1, b1, W2p, b2p)


def kernel(x, E1, E2, E3, W1, b1, W2, b2):
    pooled = _pool(x.astype(jnp.int32), E1, E2, E3)
    W2p = jnp.pad(W2, ((0, 0), (0, OUT_PAD - NUM_CLASSES)))
    b2p = jnp.pad(b2, (0, OUT_PAD - NUM_CLASSES)).reshape(1, OUT_PAD)
    out = _mlp(pooled, W1, b1.reshape(1, HIDDEN), W2p, b2p)
    return out[:, :NUM_CLASSES]


# speedup vs baseline: 23.0423x; 1.1851x over previous
import functools

import jax
import jax.numpy as jnp
from jax import lax
from jax.experimental import pallas as pl
from jax.experimental.pallas import tpu as pltpu
from jax.experimental.pallas import tpu_sc as plsc

BATCH = 4096
SEQ = 200
EMBED = 64
FEAT = 3 * EMBED
HIDDEN = 256
NUM_CLASSES = 10
OUT_PAD = 128

_NC = 2
_NS = 16
_NW = _NC * _NS
_RPW = BATCH // _NW

_CHUNKS = ((0, 128), (128, 72))


_NSLOT = 4
_AHEAD = 3
_NTASK = 3 * _RPW


def _pool_body(x_hbm, e1_hbm, e2_hbm, e3_hbm, out_hbm, xv,
               bv0, bv1, bv2, bv3, outv, sem0, sem1, sem2, sem3):
    wid = lax.axis_index("s") * _NC + lax.axis_index("c")
    base = wid * _RPW
    pltpu.sync_copy(x_hbm.at[pl.ds(base, _RPW)], xv)

    tables = (e1_hbm, e2_hbm, e3_hbm)
    bufs = (bv0, bv1, bv2, bv3)
    sems = (sem0, sem1, sem2, sem3)
    inv_seq = 1.0 / SEQ

    def copies(row, tab, slot):
        return [
            pltpu.make_async_copy(
                tables[tab].at[xv.at[row, pl.ds(off, cnt)]],
                bufs[slot].at[pl.ds(off, cnt)],
                sems[slot])
            for (off, cnt) in _CHUNKS
        ]

    def accumulate(row, tab, slot):
        bs = bufs[slot]

        def acc_step(i, accs):
            s = 4 * i
            new = list(accs)
            for u in range(4):
                for j in range(4):
                    new[j] = new[j] + bs[s + u, pl.ds(16 * j, 16)]
            return tuple(new)

        zero = jnp.zeros((16,), jnp.float32)
        accs = lax.fori_loop(0, SEQ // 4, acc_step, (zero,) * 4)
        for j in range(4):
            outv[row, pl.ds(64 * tab + 16 * j, 16)] = accs[j] * inv_seq

    for t in range(_AHEAD):
        for c in copies(t // 3, t % 3, t % _NSLOT):
            c.start()

    def group_step(i, carry):
        t0 = 12 * i
        for j in range(12):
            t = t0 + j
            row = 4 * i + j // 3
            tab = j % 3
            slot = j % _NSLOT
            for c in copies(row, tab, slot):
                c.wait()
            jn = j + _AHEAD
            nrow = 4 * i + jn // 3
            ntab = jn % 3
            nslot = jn % _NSLOT

            @pl.when(t + _AHEAD < _NTASK)
            def _():
                for c in copies(nrow, ntab, nslot):
                    c.start()

            accumulate(row, tab, slot)
        return carry

    lax.fori_loop(0, _NTASK // 12, group_step, 0)
    pltpu.sync_copy(outv, out_hbm.at[pl.ds(base, _RPW)])


@jax.jit
def _pool(x, E1, E2, E3):
    mesh = plsc.VectorSubcoreMesh(core_axis_name="c", subcore_axis_name="s")
    return pl.kernel(
        _pool_body,
        out_type=jax.ShapeDtypeStruct((BATCH, FEAT), jnp.float32),
        mesh=mesh,
        scratch_types=[
            pltpu.VMEM((_RPW, SEQ), jnp.int32),
            pltpu.VMEM((SEQ, EMBED), jnp.float32),
            pltpu.VMEM((SEQ, EMBED), jnp.float32),
            pltpu.VMEM((SEQ, EMBED), jnp.float32),
            pltpu.VMEM((SEQ, EMBED), jnp.float32),
            pltpu.VMEM((_RPW, FEAT), jnp.float32),
            pltpu.SemaphoreType.DMA,
            pltpu.SemaphoreType.DMA,
            pltpu.SemaphoreType.DMA,
            pltpu.SemaphoreType.DMA,
        ],
        compiler_params=pltpu.CompilerParams(use_tc_tiling_on_sc=False),
    )(x, E1, E2, E3)


def _mlp_body(p_ref, w1_ref, b1_ref, w2_ref, b2_ref, o_ref):
    h = jnp.dot(p_ref[...], w1_ref[...], preferred_element_type=jnp.float32)
    h = jnp.maximum(h + b1_ref[...], 0.0)
    o_ref[...] = (
        jnp.dot(h, w2_ref[...], preferred_element_type=jnp.float32)
        + b2_ref[...])


@jax.jit
def _mlp(pooled, W1, b1, W2p, b2p):
    blk = 256
    grid = BATCH // blk
    return pl.pallas_call(
        _mlp_body,
        grid=(grid,),
        in_specs=[
            pl.BlockSpec((blk, FEAT), lambda i: (i, 0)),
            pl.BlockSpec((FEAT, HIDDEN), lambda i: (0, 0)),
            pl.BlockSpec((1, HIDDEN), lambda i: (0, 0)),
            pl.BlockSpec((HIDDEN, OUT_PAD), lambda i: (0, 0)),
            pl.BlockSpec((1, OUT_PAD), lambda i: (0, 0)),
        ],
        out_specs=pl.BlockSpec((blk, OUT_PAD), lambda i: (i, 0)),
        out_shape=jax.ShapeDtypeStruct((BATCH, OUT_PAD), jnp.float32),
    )(pooled, W1, b1, W2p, b2p)


def kernel(x, E1, E2, E3, W1, b1, W2, b2):
    pooled = _pool(x.astype(jnp.int32), E1, E2, E3)
    W2p = jnp.pad(W2, ((0, 0), (0, OUT_PAD - NUM_CLASSES)))
    b2p = jnp.pad(b2, (0, OUT_PAD - NUM_CLASSES)).reshape(1, OUT_PAD)
    out = _mlp(pooled, W1, b1.reshape(1, HIDDEN), W2p, b2p)
    return out[:, :NUM_CLASSES]


# speedup vs baseline: 24.0001x; 1.0416x over previous
import functools

import jax
import jax.numpy as jnp
from jax import lax
from jax.experimental import pallas as pl
from jax.experimental.pallas import tpu as pltpu
from jax.experimental.pallas import tpu_sc as plsc

BATCH = 4096
SEQ = 200
EMBED = 64
HIDDEN = 256
NUM_CLASSES = 10
OUT_PAD = 128

_NC = 2
_NS = 16
_NW = _NC * _NS
_RPW = BATCH // _NW

_CHUNKS = ((0, 128), (128, 72))
_NSLOT = 4
_AHEAD = 3


def _pool_body(x_hbm, e_hbm, out_hbm, xv, bv0, bv1, bv2, bv3, outv,
               sem0, sem1, sem2, sem3):
    wid = lax.axis_index("s") * _NC + lax.axis_index("c")
    base = wid * _RPW
    pltpu.sync_copy(x_hbm.at[pl.ds(base, _RPW)], xv)

    bufs = (bv0, bv1, bv2, bv3)
    sems = (sem0, sem1, sem2, sem3)
    inv_seq = 1.0 / SEQ

    def copies(row, slot):
        return [
            pltpu.make_async_copy(
                e_hbm.at[xv.at[row, pl.ds(off, cnt)]],
                bufs[slot].at[pl.ds(off, cnt)],
                sems[slot])
            for (off, cnt) in _CHUNKS
        ]

    def accumulate(row, slot):
        bs = bufs[slot]

        def acc_step(i, accs):
            s = 4 * i
            new = list(accs)
            for u in range(4):
                for j in range(4):
                    new[j] = new[j] + bs[s + u, pl.ds(16 * j, 16)]
            return tuple(new)

        zero = jnp.zeros((16,), jnp.float32)
        accs = lax.fori_loop(0, SEQ // 4, acc_step, (zero,) * 4)
        for j in range(4):
            outv[row, pl.ds(16 * j, 16)] = accs[j] * inv_seq

    for r in range(_AHEAD):
        for c in copies(r, r % _NSLOT):
            c.start()

    def group_step(i, carry):
        for j in range(_NSLOT):
            row = _NSLOT * i + j
            for c in copies(row, j):
                c.wait()
            nrow = row + _AHEAD
            nslot = (j + _AHEAD) % _NSLOT

            @pl.when(nrow < _RPW)
            def _():
                for c in copies(nrow, nslot):
                    c.start()

            accumulate(row, j)
        return carry

    lax.fori_loop(0, _RPW // _NSLOT, group_step, 0)
    pltpu.sync_copy(outv, out_hbm.at[pl.ds(base, _RPW)])


@jax.jit
def _pool(x, E):
    mesh = plsc.VectorSubcoreMesh(core_axis_name="c", subcore_axis_name="s")
    return pl.kernel(
        _pool_body,
        out_type=jax.ShapeDtypeStruct((BATCH, EMBED), jnp.float32),
        mesh=mesh,
        scratch_types=[
            pltpu.VMEM((_RPW, SEQ), jnp.int32),
            pltpu.VMEM((SEQ, EMBED), jnp.float32),
            pltpu.VMEM((SEQ, EMBED), jnp.float32),
            pltpu.VMEM((SEQ, EMBED), jnp.float32),
            pltpu.VMEM((SEQ, EMBED), jnp.float32),
            pltpu.VMEM((_RPW, EMBED), jnp.float32),
            pltpu.SemaphoreType.DMA,
            pltpu.SemaphoreType.DMA,
            pltpu.SemaphoreType.DMA,
            pltpu.SemaphoreType.DMA,
        ],
        compiler_params=pltpu.CompilerParams(use_tc_tiling_on_sc=False),
    )(x, E)


def _mlp_body(p1_ref, p2_ref, p3_ref, w1a_ref, w1b_ref, w1c_ref, b1_ref,
              w2_ref, b2_ref, o_ref):
    h = jnp.dot(p1_ref[...], w1a_ref[...], preferred_element_type=jnp.float32)
    h += jnp.dot(p2_ref[...], w1b_ref[...], preferred_element_type=jnp.float32)
    h += jnp.dot(p3_ref[...], w1c_ref[...], preferred_element_type=jnp.float32)
    h = jnp.maximum(h + b1_ref[...], 0.0)
    o_ref[...] = (
        jnp.dot(h, w2_ref[...], preferred_element_type=jnp.float32)
        + b2_ref[...])


@jax.jit
def _mlp(p1, p2, p3, W1a, W1b, W1c, b1, W2p, b2p):
    blk = 256
    grid = BATCH // blk
    full = lambda i: (0, 0)
    return pl.pallas_call(
        _mlp_body,
        grid=(grid,),
        in_specs=[
            pl.BlockSpec((blk, EMBED), lambda i: (i, 0)),
            pl.BlockSpec((blk, EMBED), lambda i: (i, 0)),
            pl.BlockSpec((blk, EMBED), lambda i: (i, 0)),
            pl.BlockSpec((EMBED, HIDDEN), full),
            pl.BlockSpec((EMBED, HIDDEN), full),
            pl.BlockSpec((EMBED, HIDDEN), full),
            pl.BlockSpec((1, HIDDEN), full),
            pl.BlockSpec((HIDDEN, OUT_PAD), full),
            pl.BlockSpec((1, OUT_PAD), full),
        ],
        out_specs=pl.BlockSpec((blk, OUT_PAD), lambda i: (i, 0)),
        out_shape=jax.ShapeDtypeStruct((BATCH, OUT_PAD), jnp.float32),
    )(p1, p2, p3, W1a, W1b, W1c, b1, W2p, b2p)


def kernel(x, E1, E2, E3, W1, b1, W2, b2):
    xi = x.astype(jnp.int32)
    p1 = _pool(xi, E1)
    p2 = _pool(xi, E2)
    p3 = _pool(xi, E3)
    W2p = jnp.pad(W2, ((0, 0), (0, OUT_PAD - NUM_CLASSES)))
    b2p = jnp.pad(b2, (0, OUT_PAD - NUM_CLASSES)).reshape(1, OUT_PAD)
    out = _mlp(p1, p2, p3, W1[:EMBED], W1[EMBED:2 * EMBED], W1[2 * EMBED:],
               b1.reshape(1, HIDDEN), W2p, b2p)
    return out[:, :NUM_CLASSES]
